# baseline (speedup 1.0000x reference)
import functools

import jax
import jax.numpy as jnp
from jax import lax
from jax.experimental import pallas as pl
from jax.experimental.pallas import tpu as pltpu
from jax.experimental.pallas import tpu_sc as plsc

_MARGIN = 3.0
_DIM = 64
_LANES = 16
_CHUNK = 128
_NREL = 1000


def _make_sc_kernel(batch):
    info = plsc.get_sparse_core_info()
    nw = info.num_cores * info.num_subcores
    per_w = batch // nw
    n_chunks = per_w // _CHUNK
    mesh = plsc.VectorSubcoreMesh(core_axis_name="c", subcore_axis_name="s")

    @functools.partial(
        pl.kernel,
        mesh=mesh,
        out_type=jax.ShapeDtypeStruct((nw, _LANES), jnp.float32),
        compiler_params=pltpu.CompilerParams(
            use_tc_tiling_on_sc=False, needs_layout_passes=False),
        scratch_types=[
            pltpu.VMEM((_CHUNK,), jnp.int32),
            pltpu.VMEM((_CHUNK,), jnp.int32),
            pltpu.VMEM((_CHUNK,), jnp.int32),
            pltpu.VMEM((_CHUNK,), jnp.int32),
            pltpu.VMEM((_CHUNK,), jnp.int32),
            pltpu.VMEM((_CHUNK * _DIM,), jnp.float32),
            pltpu.VMEM((_CHUNK * _DIM,), jnp.float32),
            pltpu.VMEM((_CHUNK * _DIM,), jnp.float32),
            pltpu.VMEM((_CHUNK * _DIM,), jnp.float32),
            pltpu.VMEM((_NREL * _DIM,), jnp.float32),
            pltpu.VMEM((_LANES,), jnp.float32),
            pltpu.SemaphoreType.DMA,
        ],
    )
    def trans_e(ph_hbm, pr_hbm, pt_hbm, nh_hbm, nt_hbm, ent_lin, rel_lin,
                out_hbm, ph_i, pr_i, pt_i, nh_i, nt_i,
                h_rows, t_rows, nh_rows, nt_rows, rel_l, part_v, sem):
        wid = lax.axis_index("s") * info.num_cores + lax.axis_index("c")
        lane = lax.iota(jnp.int32, _LANES)
        zero16 = jnp.zeros((_LANES,), jnp.float32)

        pltpu.sync_copy(rel_lin, rel_l)

        def chunk_body(c, part):
            base = wid * per_w + c * _CHUNK
            pltpu.sync_copy(ph_hbm.at[pl.ds(base, _CHUNK)], ph_i)
            pltpu.sync_copy(pr_hbm.at[pl.ds(base, _CHUNK)], pr_i)
            pltpu.sync_copy(pt_hbm.at[pl.ds(base, _CHUNK)], pt_i)
            pltpu.sync_copy(nh_hbm.at[pl.ds(base, _CHUNK)], nh_i)
            pltpu.sync_copy(nt_hbm.at[pl.ds(base, _CHUNK)], nt_i)

            def fire_body(g, carry):
                base16 = pl.ds(g * _LANES, _LANES)
                phv, ptv = ph_i[base16] * _DIM, pt_i[base16] * _DIM
                nhv, ntv = nh_i[base16] * _DIM, nt_i[base16] * _DIM
                for u in range(_LANES):
                    dst = pl.ds((g * _LANES + u) * _DIM, _DIM)
                    src = lambda v: pl.ds(pl.multiple_of(v[u], _DIM), _DIM)
                    pltpu.async_copy(ent_lin.at[src(phv)], h_rows.at[dst], sem)
                    pltpu.async_copy(ent_lin.at[src(ptv)], t_rows.at[dst], sem)
                    pltpu.async_copy(ent_lin.at[src(nhv)], nh_rows.at[dst],
                                     sem)
                    pltpu.async_copy(ent_lin.at[src(ntv)], nt_rows.at[dst],
                                     sem)
                return carry

            lax.fori_loop(0, _CHUNK // _LANES, fire_body, 0)
            for buf in (h_rows, t_rows, nh_rows, nt_rows):
                pltpu.make_async_copy(
                    ent_lin.at[pl.ds(0, _CHUNK * _DIM)], buf, sem).wait()

            def g_body(g, part):
                row_off = (lane + g * _LANES) * _DIM
                prv = pr_i[pl.ds(g * _LANES, _LANES)] * _DIM

                def d_body(d, acc):
                    for du in range(4):
                        dd = d * 4 + du
                        idx = row_off + dd
                        rv = plsc.load_gather(rel_l, [prv + dd])
                        hv = plsc.load_gather(h_rows, [idx])
                        tv = plsc.load_gather(t_rows, [idx])
                        nhv = plsc.load_gather(nh_rows, [idx])
                        ntv = plsc.load_gather(nt_rows, [idx])
                        acc = acc + (jnp.abs(hv + rv - tv)
                                     - jnp.abs(nhv + rv - ntv))
                    return acc

                diff = lax.fori_loop(0, _DIM // 4, d_body, zero16)
                return part + jnp.maximum(diff + _MARGIN, 0.0)

            return lax.fori_loop(0, _CHUNK // _LANES, g_body, part)

        part = lax.fori_loop(0, n_chunks, chunk_body, zero16)
        part_v[...] = part
        pltpu.sync_copy(part_v, out_hbm.at[wid])

    return trans_e


@jax.jit
def kernel(pos_h, pos_r, pos_t, neg_h, neg_t, ent_emb, rel_emb):
    batch = pos_h.shape[0]
    sc_fn = _make_sc_kernel(batch)
    partials = sc_fn(pos_h.astype(jnp.int32), pos_r.astype(jnp.int32),
                     pos_t.astype(jnp.int32), neg_h.astype(jnp.int32),
                     neg_t.astype(jnp.int32),
                     ent_emb.reshape(-1), rel_emb.reshape(-1))
    return jnp.sum(partials) / batch
